# Initial kernel scaffold; baseline (speedup 1.0000x reference)
import jax
import jax.numpy as jnp
from jax import lax
from jax.experimental import pallas as pl
from jax.experimental.pallas import tpu as pltpu
from jax.experimental.pallas import tpu_sc as plsc

N_ENT = 100000
N_REL = 64
E_DIM = 128
K_NBR = 32
BATCH = 8192

NC = 2
NS = 16
NW = NC * NS
ROWS_W = BATCH // NW
T_ROWS_W = ROWS_W * K_NBR
CHUNK = 128


def _sc_mesh():
    return plsc.VectorSubcoreMesh(core_axis_name="c", subcore_axis_name="s")


def _wid():
    return lax.axis_index("s") * NC + lax.axis_index("c")


def _gather_adj_body(idx2_hbm, adj_ent_hbm, adj_rel_hbm, ent_hbm,
                     eids_hbm, rids_hbm, hraw_hbm,
                     idx_v, ea_v, er_v, h_v, sem):
    wid = _wid()
    nrow = ROWS_W // 128
    pltpu.sync_copy(idx2_hbm.at[pl.ds(wid * nrow, nrow)], idx_v)
    for j in range(nrow):
        row0 = wid * ROWS_W + j * 128
        pltpu.async_copy(adj_ent_hbm.at[idx_v.at[j]], ea_v, sem).wait()
        pltpu.sync_copy(ea_v, eids_hbm.at[pl.ds(row0, 128)])
        pltpu.async_copy(adj_rel_hbm.at[idx_v.at[j]], er_v, sem).wait()
        pltpu.sync_copy(er_v, rids_hbm.at[pl.ds(row0, 128)])
        pltpu.async_copy(ent_hbm.at[idx_v.at[j]], h_v, sem).wait()
        pltpu.sync_copy(h_v, hraw_hbm.at[pl.ds(row0, 128)])


def _gather_adj(idx2, adj_ent, adj_rel, ent_embs):
    kern = pl.kernel(
        _gather_adj_body,
        out_type=(
            jax.ShapeDtypeStruct((BATCH, K_NBR), jnp.int32),
            jax.ShapeDtypeStruct((BATCH, K_NBR), jnp.int32),
            jax.ShapeDtypeStruct((BATCH, E_DIM), jnp.float32),
        ),
        mesh=_sc_mesh(),
        scratch_types=[
            pltpu.VMEM((ROWS_W // 128, 128), jnp.int32),
            pltpu.VMEM((128, K_NBR), jnp.int32),
            pltpu.VMEM((128, K_NBR), jnp.int32),
            pltpu.VMEM((128, E_DIM), jnp.float32),
            pltpu.SemaphoreType.DMA,
        ],
    )
    return kern(idx2, adj_ent, adj_rel, ent_embs)


def _gather_t_body(e2_hbm, ent_hbm, t_hbm, idx_v, t_v, sem):
    wid = _wid()
    nidx = T_ROWS_W // 128
    pltpu.sync_copy(e2_hbm.at[pl.ds(wid * nidx, nidx)], idx_v)
    base = wid * T_ROWS_W

    def body(c, carry):
        pltpu.async_copy(ent_hbm.at[idx_v.at[c]], t_v, sem).wait()
        pltpu.sync_copy(t_v, t_hbm.at[pl.ds(base + c * CHUNK, CHUNK)])
        return carry

    lax.fori_loop(0, nidx, body, 0)


def _gather_t(e2, ent_embs):
    kern = pl.kernel(
        _gather_t_body,
        out_type=jax.ShapeDtypeStruct((BATCH * K_NBR, E_DIM), jnp.float32),
        mesh=_sc_mesh(),
        scratch_types=[
            pltpu.VMEM((T_ROWS_W // 128, 128), jnp.int32),
            pltpu.VMEM((CHUNK, E_DIM), jnp.float32),
            pltpu.SemaphoreType.DMA,
        ],
    )
    return kern(e2, ent_embs)


BB = 256


def _tc_body(t_ref, h_ref, rid_ref, rel_ref, wr_ref, wrb_ref,
             w1_ref, w1b_ref, w2_ref, w2b_ref, out_ref):
    f32 = jnp.float32

    def mx(e):
        n = jnp.sqrt(jnp.sum(e * e, axis=1, keepdims=True))
        return e * jnp.where(n > 1.0, 1.0 / jnp.maximum(n, 1e-7), 1.0)

    def dot_t(a, b):
        return lax.dot_general(a, b, (((1,), (1,)), ((), ())),
                               preferred_element_type=f32)

    hn = mx(h_ref[...])
    reln = mx(rel_ref[...])
    wrb = wrb_ref[...]
    hr = dot_t(hn, wr_ref[...]) + wrb

    iota_rel = lax.broadcasted_iota(jnp.int32, (1, N_REL), 1)
    logit_cols = []
    scale_cols = []
    for k in range(K_NBR):
        t_k = t_ref[:, k, :]
        n = jnp.sqrt(jnp.sum(t_k * t_k, axis=1, keepdims=True))
        sc = jnp.where(n > 1.0, 1.0 / jnp.maximum(n, 1e-7), 1.0)
        tn = t_k * sc
        oh = (rid_ref[:, k:k + 1] == iota_rel).astype(f32)
        re_k = jnp.dot(oh, reln, preferred_element_type=f32)
        tr_k = dot_t(tn, wr_ref[...]) + wrb
        g_k = jnp.tanh(hr + re_k)
        logit_cols.append(jnp.sum(g_k * tr_k, axis=1, keepdims=True))
        scale_cols.append(sc)

    logits = jnp.concatenate(logit_cols, axis=1)
    m = jnp.max(logits, axis=1, keepdims=True)
    e = jnp.exp(logits - m)
    att = e / jnp.sum(e, axis=1, keepdims=True)
    w = att * jnp.concatenate(scale_cols, axis=1)

    nh = jnp.zeros((BB, E_DIM), f32)
    for k in range(K_NBR):
        nh = nh + w[:, k:k + 1] * t_ref[:, k, :]

    leaky = lambda x: jnp.where(x > 0, x, 0.2 * x)
    agg1 = leaky(dot_t(hn + nh, w1_ref[...]) + w1b_ref[...])
    agg2 = leaky(dot_t(hn * nh, w2_ref[...]) + w2b_ref[...])
    out_ref[...] = agg1 + agg2


def _tc_call(t3, hraw, rids, rel_embs, wr, wrb, w1, w1b, w2, w2b):
    grid = BATCH // BB
    full = lambda i: (0, 0)
    return pl.pallas_call(
        _tc_body,
        grid=(grid,),
        in_specs=[
            pl.BlockSpec((BB, K_NBR, E_DIM), lambda i: (i, 0, 0)),
            pl.BlockSpec((BB, E_DIM), lambda i: (i, 0)),
            pl.BlockSpec((BB, K_NBR), lambda i: (i, 0)),
            pl.BlockSpec((N_REL, E_DIM), full),
            pl.BlockSpec((E_DIM, E_DIM), full),
            pl.BlockSpec((1, E_DIM), full),
            pl.BlockSpec((E_DIM, E_DIM), full),
            pl.BlockSpec((1, E_DIM), full),
            pl.BlockSpec((E_DIM, E_DIM), full),
            pl.BlockSpec((1, E_DIM), full),
        ],
        out_specs=pl.BlockSpec((BB, E_DIM), lambda i: (i, 0)),
        out_shape=jax.ShapeDtypeStruct((BATCH, E_DIM), jnp.float32),
        compiler_params=pltpu.CompilerParams(
            dimension_semantics=("arbitrary",),
        ),
    )(t3, hraw, rids, rel_embs, wr, wrb, w1, w1b, w2, w2b)


@jax.jit
def kernel(idx, adj_ent, adj_rel, ent_embs, rel_embs,
           Wr_w, Wr_b, W1_w, W1_b, W2_w, W2_b):
    idx = jnp.clip(idx.astype(jnp.int32), 0, N_ENT - 1)
    idx2 = idx.reshape(BATCH // 128, 128)
    eids, rids, hraw = _gather_adj(idx2, adj_ent, adj_rel, ent_embs)
    e2 = eids.reshape(BATCH * K_NBR // 128, 128)
    traw = _gather_t(e2, ent_embs)
    t3 = traw.reshape(BATCH, K_NBR, E_DIM)
    return _tc_call(t3, hraw, rids, rel_embs,
                    Wr_w, Wr_b.reshape(1, E_DIM),
                    W1_w, W1_b.reshape(1, E_DIM),
                    W2_w, W2_b.reshape(1, E_DIM))


# speedup vs baseline: 2.7012x; 2.7012x over previous
import jax
import jax.numpy as jnp
from jax import lax
from jax.experimental import pallas as pl
from jax.experimental.pallas import tpu as pltpu
from jax.experimental.pallas import tpu_sc as plsc

N_ENT = 100000
N_REL = 64
E_DIM = 128
K_NBR = 32
BATCH = 8192

NC = 2
NS = 16
NW = NC * NS
ROWS_W = BATCH // NW
T_ROWS_W = ROWS_W * K_NBR
CHUNK = 128


def _sc_mesh():
    return plsc.VectorSubcoreMesh(core_axis_name="c", subcore_axis_name="s")


def _wid():
    return lax.axis_index("s") * NC + lax.axis_index("c")


def _gather_adj_body(idx2_hbm, adj_ent_hbm, adj_rel_hbm, ent_hbm,
                     eids_hbm, rids_hbm, hraw_hbm,
                     idx_v, ea_v, er_v, h_v, sem):
    wid = _wid()
    nrow = ROWS_W // 128
    pltpu.sync_copy(idx2_hbm.at[pl.ds(wid * nrow, nrow)], idx_v)
    for j in range(nrow):
        row0 = wid * ROWS_W + j * 128
        pltpu.async_copy(adj_ent_hbm.at[idx_v.at[j]], ea_v, sem).wait()
        pltpu.sync_copy(ea_v, eids_hbm.at[pl.ds(row0, 128)])
        pltpu.async_copy(adj_rel_hbm.at[idx_v.at[j]], er_v, sem).wait()
        pltpu.sync_copy(er_v, rids_hbm.at[pl.ds(row0, 128)])
        pltpu.async_copy(ent_hbm.at[idx_v.at[j]], h_v, sem).wait()
        pltpu.sync_copy(h_v, hraw_hbm.at[pl.ds(row0, 128)])


def _gather_adj(idx2, adj_ent, adj_rel, ent_embs):
    kern = pl.kernel(
        _gather_adj_body,
        out_type=(
            jax.ShapeDtypeStruct((BATCH, K_NBR), jnp.int32),
            jax.ShapeDtypeStruct((BATCH, K_NBR), jnp.int32),
            jax.ShapeDtypeStruct((BATCH, E_DIM), jnp.float32),
        ),
        mesh=_sc_mesh(),
        scratch_types=[
            pltpu.VMEM((ROWS_W // 128, 128), jnp.int32),
            pltpu.VMEM((128, K_NBR), jnp.int32),
            pltpu.VMEM((128, K_NBR), jnp.int32),
            pltpu.VMEM((128, E_DIM), jnp.float32),
            pltpu.SemaphoreType.DMA,
        ],
        compiler_params=pltpu.CompilerParams(use_tc_tiling_on_sc=False),
    )
    return kern(idx2, adj_ent, adj_rel, ent_embs)


def _gather_t_body(e2_hbm, ent_hbm, t_hbm, idx_v, t_v, sem):
    wid = _wid()
    nidx = T_ROWS_W // 128
    pltpu.sync_copy(e2_hbm.at[pl.ds(wid * nidx, nidx)], idx_v)
    base = wid * T_ROWS_W

    def body(c, carry):
        pltpu.async_copy(ent_hbm.at[idx_v.at[c]], t_v, sem).wait()
        pltpu.sync_copy(t_v, t_hbm.at[pl.ds(base + c * CHUNK, CHUNK)])
        return carry

    lax.fori_loop(0, nidx, body, 0)


def _gather_t(e2, ent_embs):
    kern = pl.kernel(
        _gather_t_body,
        out_type=jax.ShapeDtypeStruct((BATCH * K_NBR, E_DIM), jnp.float32),
        mesh=_sc_mesh(),
        scratch_types=[
            pltpu.VMEM((T_ROWS_W // 128, 128), jnp.int32),
            pltpu.VMEM((CHUNK, E_DIM), jnp.float32),
            pltpu.SemaphoreType.DMA,
        ],
        compiler_params=pltpu.CompilerParams(use_tc_tiling_on_sc=False),
    )
    return kern(e2, ent_embs)


BB = 256


def _tc_body(t_ref, h_ref, rid_ref, rel_ref, wr_ref, wrb_ref,
             w1_ref, w1b_ref, w2_ref, w2b_ref, out_ref):
    f32 = jnp.float32

    def mx(e):
        n = jnp.sqrt(jnp.sum(e * e, axis=1, keepdims=True))
        return e * jnp.where(n > 1.0, 1.0 / jnp.maximum(n, 1e-7), 1.0)

    def dot_t(a, b):
        return lax.dot_general(a, b, (((1,), (1,)), ((), ())),
                               preferred_element_type=f32)

    hn = mx(h_ref[...])
    reln = mx(rel_ref[...])
    wrb = wrb_ref[...]
    hr = dot_t(hn, wr_ref[...]) + wrb

    iota_rel = lax.broadcasted_iota(jnp.int32, (1, N_REL), 1)
    logit_cols = []
    scale_cols = []
    for k in range(K_NBR):
        t_k = t_ref[:, k, :]
        n = jnp.sqrt(jnp.sum(t_k * t_k, axis=1, keepdims=True))
        sc = jnp.where(n > 1.0, 1.0 / jnp.maximum(n, 1e-7), 1.0)
        tn = t_k * sc
        oh = (rid_ref[:, k:k + 1] == iota_rel).astype(f32)
        re_k = jnp.dot(oh, reln, preferred_element_type=f32)
        tr_k = dot_t(tn, wr_ref[...]) + wrb
        g_k = jnp.tanh(hr + re_k)
        logit_cols.append(jnp.sum(g_k * tr_k, axis=1, keepdims=True))
        scale_cols.append(sc)

    logits = jnp.concatenate(logit_cols, axis=1)
    m = jnp.max(logits, axis=1, keepdims=True)
    e = jnp.exp(logits - m)
    att = e / jnp.sum(e, axis=1, keepdims=True)
    w = att * jnp.concatenate(scale_cols, axis=1)

    nh = jnp.zeros((BB, E_DIM), f32)
    for k in range(K_NBR):
        nh = nh + w[:, k:k + 1] * t_ref[:, k, :]

    leaky = lambda x: jnp.where(x > 0, x, 0.2 * x)
    agg1 = leaky(dot_t(hn + nh, w1_ref[...]) + w1b_ref[...])
    agg2 = leaky(dot_t(hn * nh, w2_ref[...]) + w2b_ref[...])
    out_ref[...] = agg1 + agg2


def _tc_call(t3, hraw, rids, rel_embs, wr, wrb, w1, w1b, w2, w2b):
    grid = BATCH // BB
    full = lambda i: (0, 0)
    return pl.pallas_call(
        _tc_body,
        grid=(grid,),
        in_specs=[
            pl.BlockSpec((BB, K_NBR, E_DIM), lambda i: (i, 0, 0)),
            pl.BlockSpec((BB, E_DIM), lambda i: (i, 0)),
            pl.BlockSpec((BB, K_NBR), lambda i: (i, 0)),
            pl.BlockSpec((N_REL, E_DIM), full),
            pl.BlockSpec((E_DIM, E_DIM), full),
            pl.BlockSpec((1, E_DIM), full),
            pl.BlockSpec((E_DIM, E_DIM), full),
            pl.BlockSpec((1, E_DIM), full),
            pl.BlockSpec((E_DIM, E_DIM), full),
            pl.BlockSpec((1, E_DIM), full),
        ],
        out_specs=pl.BlockSpec((BB, E_DIM), lambda i: (i, 0)),
        out_shape=jax.ShapeDtypeStruct((BATCH, E_DIM), jnp.float32),
        compiler_params=pltpu.CompilerParams(
            dimension_semantics=("arbitrary",),
        ),
    )(t3, hraw, rids, rel_embs, wr, wrb, w1, w1b, w2, w2b)


@jax.jit
def kernel(idx, adj_ent, adj_rel, ent_embs, rel_embs,
           Wr_w, Wr_b, W1_w, W1_b, W2_w, W2_b):
    idx = jnp.clip(idx.astype(jnp.int32), 0, N_ENT - 1)
    idx2 = idx.reshape(BATCH // 128, 128)
    eids, rids, hraw = _gather_adj(idx2, adj_ent, adj_rel, ent_embs)
    e2 = eids.reshape(BATCH * K_NBR // 128, 128)
    traw = _gather_t(e2, ent_embs)
    t3 = traw.reshape(BATCH, K_NBR, E_DIM)
    return _tc_call(t3, hraw, rids, rel_embs,
                    Wr_w, Wr_b.reshape(1, E_DIM),
                    W1_w, W1_b.reshape(1, E_DIM),
                    W2_w, W2_b.reshape(1, E_DIM))


# speedup vs baseline: 4.6538x; 1.7229x over previous
import jax
import jax.numpy as jnp
from jax import lax
from jax.experimental import pallas as pl
from jax.experimental.pallas import tpu as pltpu
from jax.experimental.pallas import tpu_sc as plsc

N_ENT = 100000
N_REL = 64
E_DIM = 128
K_NBR = 32
BATCH = 8192

NC = 2
NS = 16
NW = NC * NS
ROWS_W = BATCH // NW
T_ROWS_W = ROWS_W * K_NBR
CHUNK = 128


def _sc_mesh():
    return plsc.VectorSubcoreMesh(core_axis_name="c", subcore_axis_name="s")


def _wid():
    return lax.axis_index("s") * NC + lax.axis_index("c")


def _gather_adj_body(idx2_hbm, adj_ent_hbm, adj_rel_hbm, ent_hbm,
                     eids_hbm, rids_hbm, hraw_hbm,
                     idx_v, ea_v, er_v, h_v, sem):
    wid = _wid()
    nrow = ROWS_W // 128
    pltpu.sync_copy(idx2_hbm.at[pl.ds(wid * nrow, nrow)], idx_v)
    for j in range(nrow):
        row0 = wid * ROWS_W + j * 128
        pltpu.async_copy(adj_ent_hbm.at[idx_v.at[j]], ea_v, sem).wait()
        pltpu.sync_copy(ea_v, eids_hbm.at[pl.ds(row0, 128)])
        pltpu.async_copy(adj_rel_hbm.at[idx_v.at[j]], er_v, sem).wait()
        pltpu.sync_copy(er_v, rids_hbm.at[pl.ds(row0, 128)])
        pltpu.async_copy(ent_hbm.at[idx_v.at[j]], h_v, sem).wait()
        pltpu.sync_copy(h_v, hraw_hbm.at[pl.ds(row0, 128)])


def _gather_adj(idx2, adj_ent, adj_rel, ent_embs):
    kern = pl.kernel(
        _gather_adj_body,
        out_type=(
            jax.ShapeDtypeStruct((BATCH, K_NBR), jnp.int32),
            jax.ShapeDtypeStruct((BATCH, K_NBR), jnp.int32),
            jax.ShapeDtypeStruct((BATCH, E_DIM), jnp.float32),
        ),
        mesh=_sc_mesh(),
        scratch_types=[
            pltpu.VMEM((ROWS_W // 128, 128), jnp.int32),
            pltpu.VMEM((128, K_NBR), jnp.int32),
            pltpu.VMEM((128, K_NBR), jnp.int32),
            pltpu.VMEM((128, E_DIM), jnp.float32),
            pltpu.SemaphoreType.DMA,
        ],
        compiler_params=pltpu.CompilerParams(use_tc_tiling_on_sc=False),
    )
    return kern(idx2, adj_ent, adj_rel, ent_embs)


def _gather_t_body(e2_hbm, ent_hbm, t_hbm, idx_v, t_v, sem):
    wid = _wid()
    nidx = T_ROWS_W // 128
    pltpu.sync_copy(e2_hbm.at[pl.ds(wid * nidx, nidx)], idx_v)
    base = wid * T_ROWS_W

    def body(c, carry):
        pltpu.async_copy(ent_hbm.at[idx_v.at[c]], t_v, sem).wait()
        pltpu.sync_copy(t_v, t_hbm.at[pl.ds(base + c * CHUNK, CHUNK)])
        return carry

    lax.fori_loop(0, nidx, body, 0)


def _gather_t(e2, ent_embs):
    kern = pl.kernel(
        _gather_t_body,
        out_type=jax.ShapeDtypeStruct((BATCH * K_NBR, E_DIM), jnp.float32),
        mesh=_sc_mesh(),
        scratch_types=[
            pltpu.VMEM((T_ROWS_W // 128, 128), jnp.int32),
            pltpu.VMEM((CHUNK, E_DIM), jnp.float32),
            pltpu.SemaphoreType.DMA,
        ],
        compiler_params=pltpu.CompilerParams(use_tc_tiling_on_sc=False),
    )
    return kern(e2, ent_embs)


BB = 256


def _tc_body(t_ref, h_ref, rid_ref, rel_ref, wr_ref, wrb_ref,
             w1_ref, w1b_ref, w2_ref, w2b_ref, out_ref):
    f32 = jnp.float32

    def mx(e):
        n = jnp.sqrt(jnp.sum(e * e, axis=1, keepdims=True))
        return e * jnp.where(n > 1.0, 1.0 / jnp.maximum(n, 1e-7), 1.0)

    def dot_t(a, b):
        return lax.dot_general(a, b, (((1,), (1,)), ((), ())),
                               preferred_element_type=f32)

    hn = mx(h_ref[...])
    reln = mx(rel_ref[...])
    wrb = wrb_ref[...]
    hr = dot_t(hn, wr_ref[...]) + wrb

    t3 = t_ref[...]
    n = jnp.sqrt(jnp.sum(t3 * t3, axis=2))
    sc = jnp.where(n > 1.0, 1.0 / jnp.maximum(n, 1e-7), 1.0)
    t3n = t3 * sc[:, :, None]
    t2n = t3n.reshape(BB * K_NBR, E_DIM)

    tr = dot_t(t2n, wr_ref[...]) + wrb
    hrb = jnp.broadcast_to(hr[:, None, :], (BB, K_NBR, E_DIM))
    hrb = hrb.reshape(BB * K_NBR, E_DIM)

    iota_rel = lax.broadcasted_iota(jnp.int32, (1, 1, N_REL), 2)
    oh = (rid_ref[...][:, :, None] == iota_rel).astype(f32)
    oh = oh.reshape(BB * K_NBR, N_REL)
    re = jnp.dot(oh, reln, preferred_element_type=f32)

    g = jnp.tanh(hrb + re)
    prod = (g * tr).reshape(BB, K_NBR, E_DIM)
    logits = jnp.sum(prod, axis=2)

    m = jnp.max(logits, axis=1, keepdims=True)
    e = jnp.exp(logits - m)
    att = e / jnp.sum(e, axis=1, keepdims=True)
    wgt = att * sc
    nh = jnp.sum(t3 * wgt[:, :, None], axis=1)

    leaky = lambda x: jnp.where(x > 0, x, 0.2 * x)
    agg1 = leaky(dot_t(hn + nh, w1_ref[...]) + w1b_ref[...])
    agg2 = leaky(dot_t(hn * nh, w2_ref[...]) + w2b_ref[...])
    out_ref[...] = agg1 + agg2


def _tc_call(t3, hraw, rids, rel_embs, wr, wrb, w1, w1b, w2, w2b):
    grid = BATCH // BB
    full = lambda i: (0, 0)
    return pl.pallas_call(
        _tc_body,
        grid=(grid,),
        in_specs=[
            pl.BlockSpec((BB, K_NBR, E_DIM), lambda i: (i, 0, 0)),
            pl.BlockSpec((BB, E_DIM), lambda i: (i, 0)),
            pl.BlockSpec((BB, K_NBR), lambda i: (i, 0)),
            pl.BlockSpec((N_REL, E_DIM), full),
            pl.BlockSpec((E_DIM, E_DIM), full),
            pl.BlockSpec((1, E_DIM), full),
            pl.BlockSpec((E_DIM, E_DIM), full),
            pl.BlockSpec((1, E_DIM), full),
            pl.BlockSpec((E_DIM, E_DIM), full),
            pl.BlockSpec((1, E_DIM), full),
        ],
        out_specs=pl.BlockSpec((BB, E_DIM), lambda i: (i, 0)),
        out_shape=jax.ShapeDtypeStruct((BATCH, E_DIM), jnp.float32),
        compiler_params=pltpu.CompilerParams(
            dimension_semantics=("arbitrary",),
        ),
    )(t3, hraw, rids, rel_embs, wr, wrb, w1, w1b, w2, w2b)


@jax.jit
def kernel(idx, adj_ent, adj_rel, ent_embs, rel_embs,
           Wr_w, Wr_b, W1_w, W1_b, W2_w, W2_b):
    idx = jnp.clip(idx.astype(jnp.int32), 0, N_ENT - 1)
    idx2 = idx.reshape(BATCH // 128, 128)
    eids, rids, hraw = _gather_adj(idx2, adj_ent, adj_rel, ent_embs)
    e2 = eids.reshape(BATCH * K_NBR // 128, 128)
    traw = _gather_t(e2, ent_embs)
    t3 = traw.reshape(BATCH, K_NBR, E_DIM)
    return _tc_call(t3, hraw, rids, rel_embs,
                    Wr_w, Wr_b.reshape(1, E_DIM),
                    W1_w, W1_b.reshape(1, E_DIM),
                    W2_w, W2_b.reshape(1, E_DIM))


# speedup vs baseline: 5.1363x; 1.1037x over previous
import jax
import jax.numpy as jnp
from jax import lax
from jax.experimental import pallas as pl
from jax.experimental.pallas import tpu as pltpu
from jax.experimental.pallas import tpu_sc as plsc

N_ENT = 100000
N_REL = 64
E_DIM = 128
K_NBR = 32
BATCH = 8192

NC = 2
NS = 16
NW = NC * NS
ROWS_W = BATCH // NW
T_ROWS_W = ROWS_W * K_NBR
CHUNK = 128


def _sc_mesh():
    return plsc.VectorSubcoreMesh(core_axis_name="c", subcore_axis_name="s")


def _wid():
    return lax.axis_index("s") * NC + lax.axis_index("c")


def _gather_adj_body(idx2_hbm, adj_ent_hbm, adj_rel_hbm, ent_hbm,
                     eids_hbm, rids_hbm, hraw_hbm,
                     idx_v, ea_v, er_v, h_v, sem):
    wid = _wid()
    nrow = ROWS_W // 128
    pltpu.sync_copy(idx2_hbm.at[pl.ds(wid * nrow, nrow)], idx_v)
    for j in range(nrow):
        row0 = wid * ROWS_W + j * 128
        pltpu.async_copy(adj_ent_hbm.at[idx_v.at[j]], ea_v, sem).wait()
        pltpu.sync_copy(ea_v, eids_hbm.at[pl.ds(row0, 128)])
        pltpu.async_copy(adj_rel_hbm.at[idx_v.at[j]], er_v, sem).wait()
        pltpu.sync_copy(er_v, rids_hbm.at[pl.ds(row0, 128)])
        pltpu.async_copy(ent_hbm.at[idx_v.at[j]], h_v, sem).wait()
        pltpu.sync_copy(h_v, hraw_hbm.at[pl.ds(row0, 128)])


def _gather_adj(idx2, adj_ent, adj_rel, ent_embs):
    kern = pl.kernel(
        _gather_adj_body,
        out_type=(
            jax.ShapeDtypeStruct((BATCH, K_NBR), jnp.int32),
            jax.ShapeDtypeStruct((BATCH, K_NBR), jnp.int32),
            jax.ShapeDtypeStruct((BATCH, E_DIM), jnp.float32),
        ),
        mesh=_sc_mesh(),
        scratch_types=[
            pltpu.VMEM((ROWS_W // 128, 128), jnp.int32),
            pltpu.VMEM((128, K_NBR), jnp.int32),
            pltpu.VMEM((128, K_NBR), jnp.int32),
            pltpu.VMEM((128, E_DIM), jnp.float32),
            pltpu.SemaphoreType.DMA,
        ],
        compiler_params=pltpu.CompilerParams(use_tc_tiling_on_sc=False),
    )
    return kern(idx2, adj_ent, adj_rel, ent_embs)


NBUF = 4


def _gather_t_body(e2_hbm, ent_hbm, t_hbm, idx_v,
                   t_v0, t_v1, t_v2, t_v3,
                   gs0, gs1, gs2, gs3, ws0, ws1, ws2, ws3):
    wid = _wid()
    nidx = T_ROWS_W // CHUNK
    bufs = (t_v0, t_v1, t_v2, t_v3)
    gsems = (gs0, gs1, gs2, gs3)
    wsems = (ws0, ws1, ws2, ws3)
    pltpu.sync_copy(e2_hbm.at[pl.ds(wid * nidx, nidx)], idx_v)
    base = wid * T_ROWS_W

    for b in range(NBUF):
        pltpu.async_copy(ent_hbm.at[idx_v.at[b]], bufs[b], gsems[b])

    def body(i, carry):
        c0 = i * NBUF
        for b in range(NBUF):
            c = c0 + b
            pltpu.make_async_copy(ent_hbm.at[idx_v.at[c]], bufs[b],
                                  gsems[b]).wait()
            pltpu.async_copy(bufs[b],
                             t_hbm.at[pl.ds(base + c * CHUNK, CHUNK)],
                             wsems[b])
        for b in range(NBUF):
            cn = c0 + NBUF + b

            @pl.when(cn < nidx)
            def _():
                pltpu.make_async_copy(
                    bufs[b], t_hbm.at[pl.ds(base + (cn - NBUF) * CHUNK,
                                            CHUNK)], wsems[b]).wait()
                pltpu.async_copy(ent_hbm.at[idx_v.at[cn]], bufs[b], gsems[b])
        return carry

    lax.fori_loop(0, nidx // NBUF, body, 0)
    for b in range(NBUF):
        c = nidx - NBUF + b
        pltpu.make_async_copy(bufs[b],
                              t_hbm.at[pl.ds(base + c * CHUNK, CHUNK)],
                              wsems[b]).wait()


def _gather_t(e2, ent_embs):
    kern = pl.kernel(
        _gather_t_body,
        out_type=jax.ShapeDtypeStruct((BATCH * K_NBR, E_DIM), jnp.float32),
        mesh=_sc_mesh(),
        scratch_types=[
            pltpu.VMEM((T_ROWS_W // CHUNK, CHUNK), jnp.int32),
        ] + [pltpu.VMEM((CHUNK, E_DIM), jnp.float32) for _ in range(NBUF)]
          + [pltpu.SemaphoreType.DMA for _ in range(2 * NBUF)],
        compiler_params=pltpu.CompilerParams(use_tc_tiling_on_sc=False),
    )
    return kern(e2, ent_embs)


BB = 256


def _tc_body(t_ref, h_ref, rid_ref, rel_ref, wr_ref, wrb_ref,
             w1_ref, w1b_ref, w2_ref, w2b_ref, out_ref):
    f32 = jnp.float32

    def mx(e):
        n = jnp.sqrt(jnp.sum(e * e, axis=1, keepdims=True))
        return e * jnp.where(n > 1.0, 1.0 / jnp.maximum(n, 1e-7), 1.0)

    def dot_t(a, b):
        return lax.dot_general(a, b, (((1,), (1,)), ((), ())),
                               preferred_element_type=f32)

    hn = mx(h_ref[...])
    reln = mx(rel_ref[...])
    wrb = wrb_ref[...]
    hr = dot_t(hn, wr_ref[...]) + wrb

    t3 = t_ref[...]
    n = jnp.sqrt(jnp.sum(t3 * t3, axis=2))
    sc = jnp.where(n > 1.0, 1.0 / jnp.maximum(n, 1e-7), 1.0)
    t3n = t3 * sc[:, :, None]
    t2n = t3n.reshape(BB * K_NBR, E_DIM)

    tr = dot_t(t2n, wr_ref[...]) + wrb
    hrb = jnp.broadcast_to(hr[:, None, :], (BB, K_NBR, E_DIM))
    hrb = hrb.reshape(BB * K_NBR, E_DIM)

    iota_rel = lax.broadcasted_iota(jnp.int32, (1, 1, N_REL), 2)
    oh = (rid_ref[...][:, :, None] == iota_rel).astype(f32)
    oh = oh.reshape(BB * K_NBR, N_REL)
    re = jnp.dot(oh, reln, preferred_element_type=f32)

    g = jnp.tanh(hrb + re)
    prod = (g * tr).reshape(BB, K_NBR, E_DIM)
    logits = jnp.sum(prod, axis=2)

    m = jnp.max(logits, axis=1, keepdims=True)
    e = jnp.exp(logits - m)
    att = e / jnp.sum(e, axis=1, keepdims=True)
    wgt = att * sc
    nh = jnp.sum(t3 * wgt[:, :, None], axis=1)

    leaky = lambda x: jnp.where(x > 0, x, 0.2 * x)
    agg1 = leaky(dot_t(hn + nh, w1_ref[...]) + w1b_ref[...])
    agg2 = leaky(dot_t(hn * nh, w2_ref[...]) + w2b_ref[...])
    out_ref[...] = agg1 + agg2


def _tc_call(t3, hraw, rids, rel_embs, wr, wrb, w1, w1b, w2, w2b):
    grid = BATCH // BB
    full = lambda i: (0, 0)
    return pl.pallas_call(
        _tc_body,
        grid=(grid,),
        in_specs=[
            pl.BlockSpec((BB, K_NBR, E_DIM), lambda i: (i, 0, 0)),
            pl.BlockSpec((BB, E_DIM), lambda i: (i, 0)),
            pl.BlockSpec((BB, K_NBR), lambda i: (i, 0)),
            pl.BlockSpec((N_REL, E_DIM), full),
            pl.BlockSpec((E_DIM, E_DIM), full),
            pl.BlockSpec((1, E_DIM), full),
            pl.BlockSpec((E_DIM, E_DIM), full),
            pl.BlockSpec((1, E_DIM), full),
            pl.BlockSpec((E_DIM, E_DIM), full),
            pl.BlockSpec((1, E_DIM), full),
        ],
        out_specs=pl.BlockSpec((BB, E_DIM), lambda i: (i, 0)),
        out_shape=jax.ShapeDtypeStruct((BATCH, E_DIM), jnp.float32),
        compiler_params=pltpu.CompilerParams(
            dimension_semantics=("arbitrary",),
        ),
    )(t3, hraw, rids, rel_embs, wr, wrb, w1, w1b, w2, w2b)


@jax.jit
def kernel(idx, adj_ent, adj_rel, ent_embs, rel_embs,
           Wr_w, Wr_b, W1_w, W1_b, W2_w, W2_b):
    idx = jnp.clip(idx.astype(jnp.int32), 0, N_ENT - 1)
    idx2 = idx.reshape(BATCH // 128, 128)
    eids, rids, hraw = _gather_adj(idx2, adj_ent, adj_rel, ent_embs)
    e2 = eids.reshape(BATCH * K_NBR // 128, 128)
    traw = _gather_t(e2, ent_embs)
    t3 = traw.reshape(BATCH, K_NBR, E_DIM)
    return _tc_call(t3, hraw, rids, rel_embs,
                    Wr_w, Wr_b.reshape(1, E_DIM),
                    W1_w, W1_b.reshape(1, E_DIM),
                    W2_w, W2_b.reshape(1, E_DIM))


# speedup vs baseline: 6.0701x; 1.1818x over previous
import jax
import jax.numpy as jnp
from jax import lax
from jax.experimental import pallas as pl
from jax.experimental.pallas import tpu as pltpu
from jax.experimental.pallas import tpu_sc as plsc

N_ENT = 100000
N_REL = 64
E_DIM = 128
K_NBR = 32
BATCH = 8192

NC = 2
NS = 16
NW = NC * NS
ROWS_W = BATCH // NW
T_ROWS_W = ROWS_W * K_NBR
CHUNK = 128


def _sc_mesh():
    return plsc.VectorSubcoreMesh(core_axis_name="c", subcore_axis_name="s")


def _wid():
    return lax.axis_index("s") * NC + lax.axis_index("c")


def _gather_adj_body(idx2_hbm, adj_ent_hbm, adj_rel_hbm, ent_hbm,
                     eids_hbm, rids_hbm, hraw_hbm,
                     idx_v, ea_v, er_v, h_v, sem):
    wid = _wid()
    nrow = ROWS_W // 128
    pltpu.sync_copy(idx2_hbm.at[pl.ds(wid * nrow, nrow)], idx_v)
    for j in range(nrow):
        row0 = wid * ROWS_W + j * 128
        pltpu.async_copy(adj_ent_hbm.at[idx_v.at[j]], ea_v, sem).wait()
        pltpu.sync_copy(ea_v, eids_hbm.at[pl.ds(row0, 128)])
        pltpu.async_copy(adj_rel_hbm.at[idx_v.at[j]], er_v, sem).wait()
        pltpu.sync_copy(er_v, rids_hbm.at[pl.ds(row0, 128)])
        pltpu.async_copy(ent_hbm.at[idx_v.at[j]], h_v, sem).wait()
        pltpu.sync_copy(h_v, hraw_hbm.at[pl.ds(row0, 128)])


def _gather_adj(idx2, adj_ent, adj_rel, ent_embs):
    kern = pl.kernel(
        _gather_adj_body,
        out_type=(
            jax.ShapeDtypeStruct((BATCH, K_NBR), jnp.int32),
            jax.ShapeDtypeStruct((BATCH, K_NBR), jnp.int32),
            jax.ShapeDtypeStruct((BATCH, E_DIM), jnp.float32),
        ),
        mesh=_sc_mesh(),
        scratch_types=[
            pltpu.VMEM((ROWS_W // 128, 128), jnp.int32),
            pltpu.VMEM((128, K_NBR), jnp.int32),
            pltpu.VMEM((128, K_NBR), jnp.int32),
            pltpu.VMEM((128, E_DIM), jnp.float32),
            pltpu.SemaphoreType.DMA,
        ],
        compiler_params=pltpu.CompilerParams(use_tc_tiling_on_sc=False),
    )
    return kern(idx2, adj_ent, adj_rel, ent_embs)


NBUF = 4


def _gather_t_body(e2_hbm, ent_hbm, t_hbm, idx_v,
                   t_v0, t_v1, t_v2, t_v3,
                   gs0, gs1, gs2, gs3, ws0, ws1, ws2, ws3):
    wid = _wid()
    nidx = T_ROWS_W // CHUNK
    bufs = (t_v0, t_v1, t_v2, t_v3)
    gsems = (gs0, gs1, gs2, gs3)
    wsems = (ws0, ws1, ws2, ws3)
    pltpu.sync_copy(e2_hbm.at[pl.ds(wid * nidx, nidx)], idx_v)
    base = wid * T_ROWS_W

    for b in range(NBUF):
        pltpu.async_copy(ent_hbm.at[idx_v.at[b]], bufs[b], gsems[b])

    def body(i, carry):
        c0 = i * NBUF
        for b in range(NBUF):
            c = c0 + b
            pltpu.make_async_copy(ent_hbm.at[idx_v.at[c]], bufs[b],
                                  gsems[b]).wait()
            pltpu.async_copy(bufs[b],
                             t_hbm.at[pl.ds(base + c * CHUNK, CHUNK)],
                             wsems[b])
        for b in range(NBUF):
            cn = c0 + NBUF + b

            @pl.when(cn < nidx)
            def _():
                pltpu.make_async_copy(
                    bufs[b], t_hbm.at[pl.ds(base + (cn - NBUF) * CHUNK,
                                            CHUNK)], wsems[b]).wait()
                pltpu.async_copy(ent_hbm.at[idx_v.at[cn]], bufs[b], gsems[b])
        return carry

    lax.fori_loop(0, nidx // NBUF, body, 0)
    for b in range(NBUF):
        c = nidx - NBUF + b
        pltpu.make_async_copy(bufs[b],
                              t_hbm.at[pl.ds(base + c * CHUNK, CHUNK)],
                              wsems[b]).wait()


def _gather_t(e2, ent_embs):
    kern = pl.kernel(
        _gather_t_body,
        out_type=jax.ShapeDtypeStruct((BATCH * K_NBR, E_DIM), jnp.float32),
        mesh=_sc_mesh(),
        scratch_types=[
            pltpu.VMEM((T_ROWS_W // CHUNK, CHUNK), jnp.int32),
        ] + [pltpu.VMEM((CHUNK, E_DIM), jnp.float32) for _ in range(NBUF)]
          + [pltpu.SemaphoreType.DMA for _ in range(2 * NBUF)],
        compiler_params=pltpu.CompilerParams(use_tc_tiling_on_sc=False),
    )
    return kern(e2, ent_embs)


BB = 256


def _tc_body(t_ref, h_ref, rid_ref, rel_ref, wr_ref, wrb_ref,
             w1_ref, w1b_ref, w2_ref, w2b_ref, out_ref):
    f32 = jnp.float32

    def mx(e):
        n = jnp.sqrt(jnp.sum(e * e, axis=1, keepdims=True))
        return e * jnp.where(n > 1.0, 1.0 / jnp.maximum(n, 1e-7), 1.0)

    def dot_t(a, b):
        return lax.dot_general(a, b, (((1,), (1,)), ((), ())),
                               preferred_element_type=f32)

    hn = mx(h_ref[...])
    reln = mx(rel_ref[...])
    wrb = wrb_ref[...]
    hr = dot_t(hn, wr_ref[...]) + wrb

    t2 = t_ref[...].reshape(BB * K_NBR, E_DIM)
    ones_e = jnp.ones((E_DIM, E_DIM), f32)
    n2b = jnp.dot(t2 * t2, ones_e, preferred_element_type=f32)
    scb = jnp.where(n2b > 1.0, lax.rsqrt(n2b), 1.0)
    t2n = t2 * scb
    t3n = t2n.reshape(BB, K_NBR, E_DIM)

    tr = dot_t(t2n, wr_ref[...]) + wrb
    hrb = jnp.broadcast_to(hr[:, None, :], (BB, K_NBR, E_DIM))
    hrb = hrb.reshape(BB * K_NBR, E_DIM)

    iota_rel = lax.broadcasted_iota(jnp.int32, (1, 1, N_REL), 2)
    oh = (rid_ref[...][:, :, None] == iota_rel).astype(f32)
    oh = oh.reshape(BB * K_NBR, N_REL)
    re = jnp.dot(oh, reln, preferred_element_type=f32)

    g = jnp.tanh(hrb + re)
    prod = (g * tr).reshape(BB, K_NBR, E_DIM)
    logits = jnp.sum(prod, axis=2)

    lt = logits.T
    m = jnp.max(lt, axis=0, keepdims=True)
    e = jnp.exp(lt - m)
    attt = e / jnp.sum(e, axis=0, keepdims=True)
    att = attt.T
    nh = jnp.sum(t3n * att[:, :, None], axis=1)

    leaky = lambda x: jnp.where(x > 0, x, 0.2 * x)
    agg1 = leaky(dot_t(hn + nh, w1_ref[...]) + w1b_ref[...])
    agg2 = leaky(dot_t(hn * nh, w2_ref[...]) + w2b_ref[...])
    out_ref[...] = agg1 + agg2


def _tc_call(t3, hraw, rids, rel_embs, wr, wrb, w1, w1b, w2, w2b):
    grid = BATCH // BB
    full = lambda i: (0, 0)
    return pl.pallas_call(
        _tc_body,
        grid=(grid,),
        in_specs=[
            pl.BlockSpec((BB, K_NBR, E_DIM), lambda i: (i, 0, 0)),
            pl.BlockSpec((BB, E_DIM), lambda i: (i, 0)),
            pl.BlockSpec((BB, K_NBR), lambda i: (i, 0)),
            pl.BlockSpec((N_REL, E_DIM), full),
            pl.BlockSpec((E_DIM, E_DIM), full),
            pl.BlockSpec((1, E_DIM), full),
            pl.BlockSpec((E_DIM, E_DIM), full),
            pl.BlockSpec((1, E_DIM), full),
            pl.BlockSpec((E_DIM, E_DIM), full),
            pl.BlockSpec((1, E_DIM), full),
        ],
        out_specs=pl.BlockSpec((BB, E_DIM), lambda i: (i, 0)),
        out_shape=jax.ShapeDtypeStruct((BATCH, E_DIM), jnp.float32),
        compiler_params=pltpu.CompilerParams(
            dimension_semantics=("arbitrary",),
        ),
    )(t3, hraw, rids, rel_embs, wr, wrb, w1, w1b, w2, w2b)


@jax.jit
def kernel(idx, adj_ent, adj_rel, ent_embs, rel_embs,
           Wr_w, Wr_b, W1_w, W1_b, W2_w, W2_b):
    idx = jnp.clip(idx.astype(jnp.int32), 0, N_ENT - 1)
    idx2 = idx.reshape(BATCH // 128, 128)
    eids, rids, hraw = _gather_adj(idx2, adj_ent, adj_rel, ent_embs)
    e2 = eids.reshape(BATCH * K_NBR // 128, 128)
    traw = _gather_t(e2, ent_embs)
    t3 = traw.reshape(BATCH, K_NBR, E_DIM)
    return _tc_call(t3, hraw, rids, rel_embs,
                    Wr_w, Wr_b.reshape(1, E_DIM),
                    W1_w, W1_b.reshape(1, E_DIM),
                    W2_w, W2_b.reshape(1, E_DIM))


# speedup vs baseline: 6.4312x; 1.0595x over previous
import jax
import jax.numpy as jnp
from jax import lax
from jax.experimental import pallas as pl
from jax.experimental.pallas import tpu as pltpu
from jax.experimental.pallas import tpu_sc as plsc

N_ENT = 100000
N_REL = 64
E_DIM = 128
K_NBR = 32
BATCH = 8192

NC = 2
NS = 16
NW = NC * NS
ROWS_W = BATCH // NW
T_ROWS_W = ROWS_W * K_NBR
CHUNK = 128


def _sc_mesh():
    return plsc.VectorSubcoreMesh(core_axis_name="c", subcore_axis_name="s")


def _wid():
    return lax.axis_index("s") * NC + lax.axis_index("c")


def _gather_adj_body(idx2_hbm, adj_ent_hbm, adj_rel_hbm, ent_hbm,
                     eids_hbm, rids_hbm, hraw_hbm,
                     idx_v, ea_v, er_v, h_v, sem):
    wid = _wid()
    nrow = ROWS_W // 128
    pltpu.sync_copy(idx2_hbm.at[pl.ds(wid * nrow, nrow)], idx_v)
    for j in range(nrow):
        row0 = wid * ROWS_W + j * 128
        pltpu.async_copy(adj_ent_hbm.at[idx_v.at[j]], ea_v, sem).wait()
        pltpu.sync_copy(ea_v, eids_hbm.at[pl.ds(row0, 128)])
        pltpu.async_copy(adj_rel_hbm.at[idx_v.at[j]], er_v, sem).wait()
        pltpu.sync_copy(er_v, rids_hbm.at[pl.ds(row0, 128)])
        pltpu.async_copy(ent_hbm.at[idx_v.at[j]], h_v, sem).wait()
        pltpu.sync_copy(h_v, hraw_hbm.at[pl.ds(row0, 128)])


def _gather_adj(idx2, adj_ent, adj_rel, ent_embs):
    kern = pl.kernel(
        _gather_adj_body,
        out_type=(
            jax.ShapeDtypeStruct((BATCH, K_NBR), jnp.int32),
            jax.ShapeDtypeStruct((BATCH, K_NBR), jnp.int32),
            jax.ShapeDtypeStruct((BATCH, E_DIM), jnp.float32),
        ),
        mesh=_sc_mesh(),
        scratch_types=[
            pltpu.VMEM((ROWS_W // 128, 128), jnp.int32),
            pltpu.VMEM((128, K_NBR), jnp.int32),
            pltpu.VMEM((128, K_NBR), jnp.int32),
            pltpu.VMEM((128, E_DIM), jnp.float32),
            pltpu.SemaphoreType.DMA,
        ],
        compiler_params=pltpu.CompilerParams(use_tc_tiling_on_sc=False),
    )
    return kern(idx2, adj_ent, adj_rel, ent_embs)


NBUF = 4


def _make_gather_t_body(trw):
    nidx = trw // CHUNK

    def body_fn(e2_hbm, ent_hbm, t_hbm, idx_v,
                t_v0, t_v1, t_v2, t_v3,
                gs0, gs1, gs2, gs3, ws0, ws1, ws2, ws3):
        wid = _wid()
        bufs = (t_v0, t_v1, t_v2, t_v3)
        gsems = (gs0, gs1, gs2, gs3)
        wsems = (ws0, ws1, ws2, ws3)
        pltpu.sync_copy(e2_hbm.at[pl.ds(wid * nidx, nidx)], idx_v)
        base = wid * trw

        for b in range(NBUF):
            pltpu.async_copy(ent_hbm.at[idx_v.at[b]], bufs[b], gsems[b])

        def body(i, carry):
            c0 = i * NBUF
            for b in range(NBUF):
                c = c0 + b
                pltpu.make_async_copy(ent_hbm.at[idx_v.at[c]], bufs[b],
                                      gsems[b]).wait()
                pltpu.async_copy(bufs[b],
                                 t_hbm.at[pl.ds(base + c * CHUNK, CHUNK)],
                                 wsems[b])
            for b in range(NBUF):
                cn = c0 + NBUF + b

                @pl.when(cn < nidx)
                def _():
                    pltpu.make_async_copy(
                        bufs[b], t_hbm.at[pl.ds(base + (cn - NBUF) * CHUNK,
                                                CHUNK)], wsems[b]).wait()
                    pltpu.async_copy(ent_hbm.at[idx_v.at[cn]], bufs[b],
                                     gsems[b])
            return carry

        lax.fori_loop(0, nidx // NBUF, body, 0)
        for b in range(NBUF):
            c = nidx - NBUF + b
            pltpu.make_async_copy(bufs[b],
                                  t_hbm.at[pl.ds(base + c * CHUNK, CHUNK)],
                                  wsems[b]).wait()

    return body_fn


def _gather_t(e2, ent_embs, sb):
    trw = sb * K_NBR // NW
    kern = pl.kernel(
        _make_gather_t_body(trw),
        out_type=jax.ShapeDtypeStruct((sb * K_NBR, E_DIM), jnp.float32),
        mesh=_sc_mesh(),
        scratch_types=[
            pltpu.VMEM((trw // CHUNK, CHUNK), jnp.int32),
        ] + [pltpu.VMEM((CHUNK, E_DIM), jnp.float32) for _ in range(NBUF)]
          + [pltpu.SemaphoreType.DMA for _ in range(2 * NBUF)],
        compiler_params=pltpu.CompilerParams(use_tc_tiling_on_sc=False),
    )
    return kern(e2, ent_embs)


BB = 256


def _tc_body(t_ref, h_ref, rid_ref, rel_ref, wr_ref, wrb_ref,
             w1_ref, w1b_ref, w2_ref, w2b_ref, out_ref):
    f32 = jnp.float32

    def mx(e):
        n = jnp.sqrt(jnp.sum(e * e, axis=1, keepdims=True))
        return e * jnp.where(n > 1.0, 1.0 / jnp.maximum(n, 1e-7), 1.0)

    def dot_t(a, b):
        return lax.dot_general(a, b, (((1,), (1,)), ((), ())),
                               preferred_element_type=f32)

    hn = mx(h_ref[...])
    reln = mx(rel_ref[...])
    wrb = wrb_ref[...]
    hr = dot_t(hn, wr_ref[...]) + wrb

    t2 = t_ref[...].reshape(BB * K_NBR, E_DIM)
    ones_e = jnp.ones((E_DIM, E_DIM), f32)
    n2b = jnp.dot(t2 * t2, ones_e, preferred_element_type=f32)
    scb = jnp.where(n2b > 1.0, lax.rsqrt(n2b), 1.0)
    t2n = t2 * scb
    t3n = t2n.reshape(BB, K_NBR, E_DIM)

    tr = dot_t(t2n, wr_ref[...]) + wrb
    hrb = jnp.broadcast_to(hr[:, None, :], (BB, K_NBR, E_DIM))
    hrb = hrb.reshape(BB * K_NBR, E_DIM)

    iota_rel = lax.broadcasted_iota(jnp.int32, (1, 1, N_REL), 2)
    oh = (rid_ref[...][:, :, None] == iota_rel).astype(f32)
    oh = oh.reshape(BB * K_NBR, N_REL)
    re = jnp.dot(oh, reln, preferred_element_type=f32)

    g = jnp.tanh(hrb + re)
    prod = (g * tr).reshape(BB, K_NBR, E_DIM)
    logits = jnp.sum(prod, axis=2)

    lt = logits.T
    m = jnp.max(lt, axis=0, keepdims=True)
    e = jnp.exp(lt - m)
    attt = e / jnp.sum(e, axis=0, keepdims=True)
    att = attt.T
    nh = jnp.sum(t3n * att[:, :, None], axis=1)

    leaky = lambda x: jnp.where(x > 0, x, 0.2 * x)
    agg1 = leaky(dot_t(hn + nh, w1_ref[...]) + w1b_ref[...])
    agg2 = leaky(dot_t(hn * nh, w2_ref[...]) + w2b_ref[...])
    out_ref[...] = agg1 + agg2


def _tc_call(t3, hraw, rids, rel_embs, wr, wrb, w1, w1b, w2, w2b):
    grid = t3.shape[0] // BB
    full = lambda i: (0, 0)
    return pl.pallas_call(
        _tc_body,
        grid=(grid,),
        in_specs=[
            pl.BlockSpec((BB, K_NBR, E_DIM), lambda i: (i, 0, 0)),
            pl.BlockSpec((BB, E_DIM), lambda i: (i, 0)),
            pl.BlockSpec((BB, K_NBR), lambda i: (i, 0)),
            pl.BlockSpec((N_REL, E_DIM), full),
            pl.BlockSpec((E_DIM, E_DIM), full),
            pl.BlockSpec((1, E_DIM), full),
            pl.BlockSpec((E_DIM, E_DIM), full),
            pl.BlockSpec((1, E_DIM), full),
            pl.BlockSpec((E_DIM, E_DIM), full),
            pl.BlockSpec((1, E_DIM), full),
        ],
        out_specs=pl.BlockSpec((BB, E_DIM), lambda i: (i, 0)),
        out_shape=jax.ShapeDtypeStruct((t3.shape[0], E_DIM), jnp.float32),
        compiler_params=pltpu.CompilerParams(
            dimension_semantics=("arbitrary",),
        ),
    )(t3, hraw, rids, rel_embs, wr, wrb, w1, w1b, w2, w2b)


SEG = 2


@jax.jit
def kernel(idx, adj_ent, adj_rel, ent_embs, rel_embs,
           Wr_w, Wr_b, W1_w, W1_b, W2_w, W2_b):
    idx = jnp.clip(idx.astype(jnp.int32), 0, N_ENT - 1)
    idx2 = idx.reshape(BATCH // 128, 128)
    eids, rids, hraw = _gather_adj(idx2, adj_ent, adj_rel, ent_embs)
    e2 = eids.reshape(BATCH * K_NBR // 128, 128)
    sb = BATCH // SEG
    er = sb * K_NBR // 128
    outs = []
    for s in range(SEG):
        traw = _gather_t(e2[s * er:(s + 1) * er], ent_embs, sb)
        t3 = traw.reshape(sb, K_NBR, E_DIM)
        outs.append(_tc_call(t3, hraw[s * sb:(s + 1) * sb],
                             rids[s * sb:(s + 1) * sb], rel_embs,
                             Wr_w, Wr_b.reshape(1, E_DIM),
                             W1_w, W1_b.reshape(1, E_DIM),
                             W2_w, W2_b.reshape(1, E_DIM)))
    return jnp.concatenate(outs, axis=0) if SEG > 1 else outs[0]


# speedup vs baseline: 6.8068x; 1.0584x over previous
import jax
import jax.numpy as jnp
from jax import lax
from jax.experimental import pallas as pl
from jax.experimental.pallas import tpu as pltpu
from jax.experimental.pallas import tpu_sc as plsc

N_ENT = 100000
N_REL = 64
E_DIM = 128
K_NBR = 32
BATCH = 8192

NC = 2
NS = 16
NW = NC * NS
ROWS_W = BATCH // NW
T_ROWS_W = ROWS_W * K_NBR
CHUNK = 128


def _sc_mesh():
    return plsc.VectorSubcoreMesh(core_axis_name="c", subcore_axis_name="s")


def _wid():
    return lax.axis_index("s") * NC + lax.axis_index("c")


def _gather_adj_body(idx2_hbm, adj_ent_hbm, adj_rel_hbm, ent_hbm,
                     eids_hbm, rids_hbm, hraw_hbm,
                     idx_v, ea_v, er_v, h_v, sem):
    wid = _wid()
    nrow = ROWS_W // 128
    pltpu.sync_copy(idx2_hbm.at[pl.ds(wid * nrow, nrow)], idx_v)
    for j in range(nrow):
        row0 = wid * ROWS_W + j * 128
        pltpu.async_copy(adj_ent_hbm.at[idx_v.at[j]], ea_v, sem).wait()
        pltpu.sync_copy(ea_v, eids_hbm.at[pl.ds(row0, 128)])
        pltpu.async_copy(adj_rel_hbm.at[idx_v.at[j]], er_v, sem).wait()
        pltpu.sync_copy(er_v, rids_hbm.at[pl.ds(row0, 128)])
        pltpu.async_copy(ent_hbm.at[idx_v.at[j]], h_v, sem).wait()
        pltpu.sync_copy(h_v, hraw_hbm.at[pl.ds(row0, 128)])


def _gather_adj(idx2, adj_ent, adj_rel, ent_embs):
    kern = pl.kernel(
        _gather_adj_body,
        out_type=(
            jax.ShapeDtypeStruct((BATCH, K_NBR), jnp.int32),
            jax.ShapeDtypeStruct((BATCH, K_NBR), jnp.int32),
            jax.ShapeDtypeStruct((BATCH, E_DIM), jnp.float32),
        ),
        mesh=_sc_mesh(),
        scratch_types=[
            pltpu.VMEM((ROWS_W // 128, 128), jnp.int32),
            pltpu.VMEM((128, K_NBR), jnp.int32),
            pltpu.VMEM((128, K_NBR), jnp.int32),
            pltpu.VMEM((128, E_DIM), jnp.float32),
            pltpu.SemaphoreType.DMA,
        ],
        compiler_params=pltpu.CompilerParams(use_tc_tiling_on_sc=False),
    )
    return kern(idx2, adj_ent, adj_rel, ent_embs)


NBUF = 4


def _make_gather_t_body(trw):
    nidx = trw // CHUNK

    def body_fn(e2_hbm, ent_hbm, t_hbm, idx_v,
                t_v0, t_v1, t_v2, t_v3,
                gs0, gs1, gs2, gs3, ws0, ws1, ws2, ws3):
        wid = _wid()
        bufs = (t_v0, t_v1, t_v2, t_v3)
        gsems = (gs0, gs1, gs2, gs3)
        wsems = (ws0, ws1, ws2, ws3)
        pltpu.sync_copy(e2_hbm.at[pl.ds(wid * nidx, nidx)], idx_v)
        base = wid * trw

        for b in range(NBUF):
            pltpu.async_copy(ent_hbm.at[idx_v.at[b]], bufs[b], gsems[b])

        def body(i, carry):
            c0 = i * NBUF
            for b in range(NBUF):
                c = c0 + b
                pltpu.make_async_copy(ent_hbm.at[idx_v.at[c]], bufs[b],
                                      gsems[b]).wait()
                pltpu.async_copy(bufs[b],
                                 t_hbm.at[pl.ds(base + c * CHUNK, CHUNK)],
                                 wsems[b])
            for b in range(NBUF):
                cn = c0 + NBUF + b

                @pl.when(cn < nidx)
                def _():
                    pltpu.make_async_copy(
                        bufs[b], t_hbm.at[pl.ds(base + (cn - NBUF) * CHUNK,
                                                CHUNK)], wsems[b]).wait()
                    pltpu.async_copy(ent_hbm.at[idx_v.at[cn]], bufs[b],
                                     gsems[b])
            return carry

        lax.fori_loop(0, nidx // NBUF, body, 0)
        for b in range(NBUF):
            c = nidx - NBUF + b
            pltpu.make_async_copy(bufs[b],
                                  t_hbm.at[pl.ds(base + c * CHUNK, CHUNK)],
                                  wsems[b]).wait()

    return body_fn


def _gather_t(e2, ent_embs, sb):
    trw = sb * K_NBR // NW
    kern = pl.kernel(
        _make_gather_t_body(trw),
        out_type=jax.ShapeDtypeStruct((sb * K_NBR, E_DIM), jnp.float32),
        mesh=_sc_mesh(),
        scratch_types=[
            pltpu.VMEM((trw // CHUNK, CHUNK), jnp.int32),
        ] + [pltpu.VMEM((CHUNK, E_DIM), jnp.float32) for _ in range(NBUF)]
          + [pltpu.SemaphoreType.DMA for _ in range(2 * NBUF)],
        compiler_params=pltpu.CompilerParams(use_tc_tiling_on_sc=False),
    )
    return kern(e2, ent_embs)


BB = 256


def _tc_body(t_ref, h_ref, rid_ref, rel_ref, wr_ref, wrb_ref,
             w1_ref, w1b_ref, w2_ref, w2b_ref, out_ref):
    f32 = jnp.float32

    def mx(e):
        n = jnp.sqrt(jnp.sum(e * e, axis=1, keepdims=True))
        return e * jnp.where(n > 1.0, 1.0 / jnp.maximum(n, 1e-7), 1.0)

    def dot_t(a, b):
        return lax.dot_general(a, b, (((1,), (1,)), ((), ())),
                               preferred_element_type=f32)

    hn = mx(h_ref[...])
    reln = mx(rel_ref[...])
    wrb = wrb_ref[...]
    hr = dot_t(hn, wr_ref[...]) + wrb

    t2 = t_ref[...].reshape(BB * K_NBR, E_DIM)
    ones_e = jnp.ones((E_DIM, E_DIM), f32)
    n2b = jnp.dot(t2 * t2, ones_e, preferred_element_type=f32)
    scb = lax.rsqrt(jnp.maximum(n2b, 1.0))
    t2n = t2 * scb
    t3n = t2n.reshape(BB, K_NBR, E_DIM)

    tr = dot_t(t2n, wr_ref[...]) + wrb
    hrb = jnp.broadcast_to(hr[:, None, :], (BB, K_NBR, E_DIM))
    hrb = hrb.reshape(BB * K_NBR, E_DIM)

    iota_rel = lax.broadcasted_iota(jnp.int32, (1, 1, N_REL), 2)
    oh = (rid_ref[...][:, :, None] == iota_rel).astype(f32)
    oh = oh.reshape(BB * K_NBR, N_REL)
    re = jnp.dot(oh, reln, preferred_element_type=f32)

    g = jnp.tanh(hrb + re)
    prod = (g * tr).reshape(BB, K_NBR, E_DIM)
    logits = jnp.sum(prod, axis=2)

    lt = logits.T
    m = jnp.max(lt, axis=0, keepdims=True)
    e = jnp.exp(lt - m)
    attt = e / jnp.sum(e, axis=0, keepdims=True)
    att = attt.T
    nh = jnp.sum(t3n * att[:, :, None], axis=1)

    leaky = lambda x: jnp.where(x > 0, x, 0.2 * x)
    agg1 = leaky(dot_t(hn + nh, w1_ref[...]) + w1b_ref[...])
    agg2 = leaky(dot_t(hn * nh, w2_ref[...]) + w2b_ref[...])
    out_ref[...] = agg1 + agg2


def _tc_call(t3, hraw, rids, rel_embs, wr, wrb, w1, w1b, w2, w2b):
    grid = t3.shape[0] // BB
    full = lambda i: (0, 0)
    return pl.pallas_call(
        _tc_body,
        grid=(grid,),
        in_specs=[
            pl.BlockSpec((BB, K_NBR, E_DIM), lambda i: (i, 0, 0)),
            pl.BlockSpec((BB, E_DIM), lambda i: (i, 0)),
            pl.BlockSpec((BB, K_NBR), lambda i: (i, 0)),
            pl.BlockSpec((N_REL, E_DIM), full),
            pl.BlockSpec((E_DIM, E_DIM), full),
            pl.BlockSpec((1, E_DIM), full),
            pl.BlockSpec((E_DIM, E_DIM), full),
            pl.BlockSpec((1, E_DIM), full),
            pl.BlockSpec((E_DIM, E_DIM), full),
            pl.BlockSpec((1, E_DIM), full),
        ],
        out_specs=pl.BlockSpec((BB, E_DIM), lambda i: (i, 0)),
        out_shape=jax.ShapeDtypeStruct((t3.shape[0], E_DIM), jnp.float32),
        compiler_params=pltpu.CompilerParams(
            dimension_semantics=("arbitrary",),
        ),
    )(t3, hraw, rids, rel_embs, wr, wrb, w1, w1b, w2, w2b)


SEG = 4


@jax.jit
def kernel(idx, adj_ent, adj_rel, ent_embs, rel_embs,
           Wr_w, Wr_b, W1_w, W1_b, W2_w, W2_b):
    idx = jnp.clip(idx.astype(jnp.int32), 0, N_ENT - 1)
    idx2 = idx.reshape(BATCH // 128, 128)
    eids, rids, hraw = _gather_adj(idx2, adj_ent, adj_rel, ent_embs)
    e2 = eids.reshape(BATCH * K_NBR // 128, 128)
    sb = BATCH // SEG
    er = sb * K_NBR // 128
    outs = []
    for s in range(SEG):
        traw = _gather_t(e2[s * er:(s + 1) * er], ent_embs, sb)
        t3 = traw.reshape(sb, K_NBR, E_DIM)
        outs.append(_tc_call(t3, hraw[s * sb:(s + 1) * sb],
                             rids[s * sb:(s + 1) * sb], rel_embs,
                             Wr_w, Wr_b.reshape(1, E_DIM),
                             W1_w, W1_b.reshape(1, E_DIM),
                             W2_w, W2_b.reshape(1, E_DIM)))
    return jnp.concatenate(outs, axis=0) if SEG > 1 else outs[0]


# speedup vs baseline: 6.9498x; 1.0210x over previous
import jax
import jax.numpy as jnp
from jax import lax
from jax.experimental import pallas as pl
from jax.experimental.pallas import tpu as pltpu
from jax.experimental.pallas import tpu_sc as plsc

N_ENT = 100000
N_REL = 64
E_DIM = 128
K_NBR = 32
BATCH = 8192

NC = 2
NS = 16
NW = NC * NS
ROWS_W = BATCH // NW
T_ROWS_W = ROWS_W * K_NBR
CHUNK = 128


def _sc_mesh():
    return plsc.VectorSubcoreMesh(core_axis_name="c", subcore_axis_name="s")


def _wid():
    return lax.axis_index("s") * NC + lax.axis_index("c")


def _gather_adj_body(idx2_hbm, adj_ent_hbm, adj_rel_hbm,
                     eids_hbm, rids_hbm,
                     idx_v, ea_v, er_v, sem):
    wid = _wid()
    nrow = ROWS_W // 128
    pltpu.sync_copy(idx2_hbm.at[pl.ds(wid * nrow, nrow)], idx_v)
    for j in range(nrow):
        row0 = wid * ROWS_W + j * 128
        pltpu.async_copy(adj_ent_hbm.at[idx_v.at[j]], ea_v, sem).wait()
        pltpu.sync_copy(ea_v, eids_hbm.at[pl.ds(row0, 128)])
        pltpu.async_copy(adj_rel_hbm.at[idx_v.at[j]], er_v, sem).wait()
        pltpu.sync_copy(er_v, rids_hbm.at[pl.ds(row0, 128)])


def _gather_adj(idx2, adj_ent, adj_rel):
    kern = pl.kernel(
        _gather_adj_body,
        out_type=(
            jax.ShapeDtypeStruct((BATCH, K_NBR), jnp.int32),
            jax.ShapeDtypeStruct((BATCH, K_NBR), jnp.int32),
        ),
        mesh=_sc_mesh(),
        scratch_types=[
            pltpu.VMEM((ROWS_W // 128, 128), jnp.int32),
            pltpu.VMEM((128, K_NBR), jnp.int32),
            pltpu.VMEM((128, K_NBR), jnp.int32),
            pltpu.SemaphoreType.DMA,
        ],
        compiler_params=pltpu.CompilerParams(use_tc_tiling_on_sc=False),
    )
    return kern(idx2, adj_ent, adj_rel)


NBUF = 4


def _make_gather_t_body(sb, seg):
    trw = sb * K_NBR // NW
    nidx = trw // CHUNK
    hseg = sb // 128

    def body_fn(e2_hbm, idx2_hbm, ent_hbm, t_hbm, h_hbm, idx_v, hi_v, h_v,
                t_v0, t_v1, t_v2, t_v3,
                gs0, gs1, gs2, gs3, ws0, ws1, ws2, ws3, hs):
        wid = _wid()
        bufs = (t_v0, t_v1, t_v2, t_v3)
        gsems = (gs0, gs1, gs2, gs3)
        wsems = (ws0, ws1, ws2, ws3)
        pltpu.sync_copy(e2_hbm.at[pl.ds(seg * (sb * K_NBR // 128)
                                        + wid * nidx, nidx)], idx_v)
        base = wid * trw

        @pl.when(wid < hseg)
        def _():
            pltpu.sync_copy(idx2_hbm.at[pl.ds(seg * hseg + wid, 1)], hi_v)
            pltpu.async_copy(ent_hbm.at[hi_v.at[0]], h_v, hs).wait()
            pltpu.sync_copy(h_v, h_hbm.at[pl.ds(wid * 128, 128)])

        for b in range(NBUF):
            pltpu.async_copy(ent_hbm.at[idx_v.at[b]], bufs[b], gsems[b])

        def body(i, carry):
            c0 = i * NBUF
            for b in range(NBUF):
                c = c0 + b
                pltpu.make_async_copy(ent_hbm.at[idx_v.at[c]], bufs[b],
                                      gsems[b]).wait()
                pltpu.async_copy(bufs[b],
                                 t_hbm.at[pl.ds(base + c * CHUNK, CHUNK)],
                                 wsems[b])
            for b in range(NBUF):
                cn = c0 + NBUF + b

                @pl.when(cn < nidx)
                def _():
                    pltpu.make_async_copy(
                        bufs[b], t_hbm.at[pl.ds(base + (cn - NBUF) * CHUNK,
                                                CHUNK)], wsems[b]).wait()
                    pltpu.async_copy(ent_hbm.at[idx_v.at[cn]], bufs[b],
                                     gsems[b])
            return carry

        lax.fori_loop(0, nidx // NBUF, body, 0)
        for b in range(NBUF):
            c = nidx - NBUF + b
            pltpu.make_async_copy(bufs[b],
                                  t_hbm.at[pl.ds(base + c * CHUNK, CHUNK)],
                                  wsems[b]).wait()

    return body_fn


def _gather_t(e2, idx2, ent_embs, sb, seg):
    trw = sb * K_NBR // NW
    kern = pl.kernel(
        _make_gather_t_body(sb, seg),
        out_type=(
            jax.ShapeDtypeStruct((sb * K_NBR, E_DIM), jnp.float32),
            jax.ShapeDtypeStruct((sb, E_DIM), jnp.float32),
        ),
        mesh=_sc_mesh(),
        scratch_types=[
            pltpu.VMEM((trw // CHUNK, CHUNK), jnp.int32),
            pltpu.VMEM((1, 128), jnp.int32),
            pltpu.VMEM((128, E_DIM), jnp.float32),
        ] + [pltpu.VMEM((CHUNK, E_DIM), jnp.float32) for _ in range(NBUF)]
          + [pltpu.SemaphoreType.DMA for _ in range(2 * NBUF + 1)],
    )
    return kern(e2, idx2, ent_embs)


BB = 256


def _tc_body(t_ref, h_ref, rid_ref, rel_ref, wr_ref, wrb_ref,
             w1_ref, w1b_ref, w2_ref, w2b_ref, out_ref):
    f32 = jnp.float32

    def mx(e):
        n = jnp.sqrt(jnp.sum(e * e, axis=1, keepdims=True))
        return e * jnp.where(n > 1.0, 1.0 / jnp.maximum(n, 1e-7), 1.0)

    def dot_t(a, b):
        return lax.dot_general(a, b, (((1,), (1,)), ((), ())),
                               preferred_element_type=f32)

    hn = mx(h_ref[...])
    reln = mx(rel_ref[...])
    wrb = wrb_ref[...]
    hr = dot_t(hn, wr_ref[...]) + wrb

    t2 = t_ref[...].reshape(BB * K_NBR, E_DIM)
    ones_e = jnp.ones((E_DIM, E_DIM), f32)
    n2b = jnp.dot(t2 * t2, ones_e, preferred_element_type=f32)
    scb = lax.rsqrt(jnp.maximum(n2b, 1.0))
    t2n = t2 * scb
    t3n = t2n.reshape(BB, K_NBR, E_DIM)

    tr = dot_t(t2n, wr_ref[...]) + wrb
    hrb = jnp.broadcast_to(hr[:, None, :], (BB, K_NBR, E_DIM))
    hrb = hrb.reshape(BB * K_NBR, E_DIM)

    iota_rel = lax.broadcasted_iota(jnp.int32, (1, 1, N_REL), 2)
    oh = (rid_ref[...][:, :, None] == iota_rel).astype(f32)
    oh = oh.reshape(BB * K_NBR, N_REL)
    re = jnp.dot(oh, reln, preferred_element_type=f32)

    g = jnp.tanh(hrb + re)
    prod = (g * tr).reshape(BB, K_NBR, E_DIM)
    logits = jnp.sum(prod, axis=2)

    lt = logits.T
    m = jnp.max(lt, axis=0, keepdims=True)
    e = jnp.exp(lt - m)
    attt = e / jnp.sum(e, axis=0, keepdims=True)
    att = attt.T
    nh = jnp.sum(t3n * att[:, :, None], axis=1)

    leaky = lambda x: jnp.where(x > 0, x, 0.2 * x)
    agg1 = leaky(dot_t(hn + nh, w1_ref[...]) + w1b_ref[...])
    agg2 = leaky(dot_t(hn * nh, w2_ref[...]) + w2b_ref[...])
    out_ref[...] = agg1 + agg2


def _tc_call(t3, hraw, rids, rel_embs, wr, wrb, w1, w1b, w2, w2b, seg):
    grid = t3.shape[0] // BB
    soff = seg * grid
    full = lambda i: (0, 0)
    return pl.pallas_call(
        _tc_body,
        grid=(grid,),
        in_specs=[
            pl.BlockSpec((BB, K_NBR, E_DIM), lambda i: (i, 0, 0)),
            pl.BlockSpec((BB, E_DIM), lambda i: (i, 0)),
            pl.BlockSpec((BB, K_NBR), lambda i: (soff + i, 0)),
            pl.BlockSpec((N_REL, E_DIM), full),
            pl.BlockSpec((E_DIM, E_DIM), full),
            pl.BlockSpec((1, E_DIM), full),
            pl.BlockSpec((E_DIM, E_DIM), full),
            pl.BlockSpec((1, E_DIM), full),
            pl.BlockSpec((E_DIM, E_DIM), full),
            pl.BlockSpec((1, E_DIM), full),
        ],
        out_specs=pl.BlockSpec((BB, E_DIM), lambda i: (i, 0)),
        out_shape=jax.ShapeDtypeStruct((t3.shape[0], E_DIM), jnp.float32),
        compiler_params=pltpu.CompilerParams(
            dimension_semantics=("arbitrary",),
        ),
    )(t3, hraw, rids, rel_embs, wr, wrb, w1, w1b, w2, w2b)


SEG = 4


@jax.jit
def kernel(idx, adj_ent, adj_rel, ent_embs, rel_embs,
           Wr_w, Wr_b, W1_w, W1_b, W2_w, W2_b):
    idx = jnp.clip(idx.astype(jnp.int32), 0, N_ENT - 1)
    idx2 = idx.reshape(BATCH // 128, 128)
    eids, rids = _gather_adj(idx2, adj_ent, adj_rel)
    e2 = eids.reshape(BATCH * K_NBR // 128, 128)
    sb = BATCH // SEG
    outs = []
    for s in range(SEG):
        traw, hraw = _gather_t(e2, idx2, ent_embs, sb, s)
        t3 = traw.reshape(sb, K_NBR, E_DIM)
        outs.append(_tc_call(t3, hraw, rids, rel_embs,
                             Wr_w, Wr_b.reshape(1, E_DIM),
                             W1_w, W1_b.reshape(1, E_DIM),
                             W2_w, W2_b.reshape(1, E_DIM), s))
    return jnp.concatenate(outs, axis=0) if SEG > 1 else outs[0]


# speedup vs baseline: 7.0280x; 1.0112x over previous
import jax
import jax.numpy as jnp
from jax import lax
from jax.experimental import pallas as pl
from jax.experimental.pallas import tpu as pltpu
from jax.experimental.pallas import tpu_sc as plsc

N_ENT = 100000
N_REL = 64
E_DIM = 128
K_NBR = 32
BATCH = 8192

NC = 2
NS = 16
NW = NC * NS
ROWS_W = BATCH // NW
T_ROWS_W = ROWS_W * K_NBR
CHUNK = 128


def _sc_mesh():
    return plsc.VectorSubcoreMesh(core_axis_name="c", subcore_axis_name="s")


def _wid():
    return lax.axis_index("s") * NC + lax.axis_index("c")


def _gather_adj_body(idx2_hbm, adj_hbm, out_hbm, idx_v, a_v, sem):
    wid = _wid()
    nrow = ROWS_W // 128
    pltpu.sync_copy(idx2_hbm.at[pl.ds(wid * nrow, nrow)], idx_v)
    for j in range(nrow):
        row0 = wid * ROWS_W + j * 128
        pltpu.async_copy(adj_hbm.at[idx_v.at[j]], a_v, sem).wait()
        pltpu.sync_copy(a_v, out_hbm.at[pl.ds(row0, 128)])


def _gather_adj(idx2, adj):
    kern = pl.kernel(
        _gather_adj_body,
        out_type=jax.ShapeDtypeStruct((BATCH, K_NBR), jnp.int32),
        mesh=_sc_mesh(),
        scratch_types=[
            pltpu.VMEM((ROWS_W // 128, 128), jnp.int32),
            pltpu.VMEM((128, K_NBR), jnp.int32),
            pltpu.SemaphoreType.DMA,
        ],
        compiler_params=pltpu.CompilerParams(use_tc_tiling_on_sc=False),
    )
    return kern(idx2, adj)


NBUF = 4


def _make_gather_t_body(sb, seg):
    trw = sb * K_NBR // NW
    nidx = trw // CHUNK
    hseg = sb // 128

    def body_fn(e2_hbm, idx2_hbm, ent_hbm, t_hbm, h_hbm, idx_v, hi_v, h_v,
                t_v0, t_v1, t_v2, t_v3,
                gs0, gs1, gs2, gs3, ws0, ws1, ws2, ws3, hs):
        wid = _wid()
        bufs = (t_v0, t_v1, t_v2, t_v3)
        gsems = (gs0, gs1, gs2, gs3)
        wsems = (ws0, ws1, ws2, ws3)
        pltpu.sync_copy(e2_hbm.at[pl.ds(seg * (sb * K_NBR // 128)
                                        + wid * nidx, nidx)], idx_v)
        base = wid * trw

        @pl.when(wid < hseg)
        def _():
            pltpu.sync_copy(idx2_hbm.at[pl.ds(seg * hseg + wid, 1)], hi_v)
            pltpu.async_copy(ent_hbm.at[hi_v.at[0]], h_v, hs).wait()
            pltpu.sync_copy(h_v, h_hbm.at[pl.ds(wid * 128, 128)])

        for b in range(NBUF):
            pltpu.async_copy(ent_hbm.at[idx_v.at[b]], bufs[b], gsems[b])

        def body(i, carry):
            c0 = i * NBUF
            for b in range(NBUF):
                c = c0 + b
                pltpu.make_async_copy(ent_hbm.at[idx_v.at[c]], bufs[b],
                                      gsems[b]).wait()
                pltpu.async_copy(bufs[b],
                                 t_hbm.at[pl.ds(base + c * CHUNK, CHUNK)],
                                 wsems[b])
            for b in range(NBUF):
                cn = c0 + NBUF + b

                @pl.when(cn < nidx)
                def _():
                    pltpu.make_async_copy(
                        bufs[b], t_hbm.at[pl.ds(base + (cn - NBUF) * CHUNK,
                                                CHUNK)], wsems[b]).wait()
                    pltpu.async_copy(ent_hbm.at[idx_v.at[cn]], bufs[b],
                                     gsems[b])
            return carry

        lax.fori_loop(0, nidx // NBUF, body, 0)
        for b in range(NBUF):
            c = nidx - NBUF + b
            pltpu.make_async_copy(bufs[b],
                                  t_hbm.at[pl.ds(base + c * CHUNK, CHUNK)],
                                  wsems[b]).wait()

    return body_fn


def _gather_t(e2, idx2, ent_embs, sb, seg):
    trw = sb * K_NBR // NW
    kern = pl.kernel(
        _make_gather_t_body(sb, seg),
        out_type=(
            jax.ShapeDtypeStruct((sb * K_NBR, E_DIM), jnp.float32),
            jax.ShapeDtypeStruct((sb, E_DIM), jnp.float32),
        ),
        mesh=_sc_mesh(),
        scratch_types=[
            pltpu.VMEM((trw // CHUNK, CHUNK), jnp.int32),
            pltpu.VMEM((1, 128), jnp.int32),
            pltpu.VMEM((128, E_DIM), jnp.float32),
        ] + [pltpu.VMEM((CHUNK, E_DIM), jnp.float32) for _ in range(NBUF)]
          + [pltpu.SemaphoreType.DMA for _ in range(2 * NBUF + 1)],
    )
    return kern(e2, idx2, ent_embs)


BB = 256


def _tc_body(t_ref, h_ref, rid_ref, rel_ref, wr_ref, wrb_ref,
             w1_ref, w1b_ref, w2_ref, w2b_ref, out_ref):
    f32 = jnp.float32

    def mx(e):
        n = jnp.sqrt(jnp.sum(e * e, axis=1, keepdims=True))
        return e * jnp.where(n > 1.0, 1.0 / jnp.maximum(n, 1e-7), 1.0)

    def dot_t(a, b):
        return lax.dot_general(a, b, (((1,), (1,)), ((), ())),
                               preferred_element_type=f32)

    hn = mx(h_ref[...])
    reln = mx(rel_ref[...])
    wrb = wrb_ref[...]
    hr = dot_t(hn, wr_ref[...]) + wrb

    t2 = t_ref[...].reshape(BB * K_NBR, E_DIM)
    ones_e = jnp.ones((E_DIM, E_DIM), f32)
    n2b = jnp.dot(t2 * t2, ones_e, preferred_element_type=f32)
    scb = lax.rsqrt(jnp.maximum(n2b, 1.0))
    t2n = t2 * scb
    t3n = t2n.reshape(BB, K_NBR, E_DIM)

    tr = dot_t(t2n, wr_ref[...]) + wrb
    hrb = jnp.broadcast_to(hr[:, None, :], (BB, K_NBR, E_DIM))
    hrb = hrb.reshape(BB * K_NBR, E_DIM)

    iota_rel = lax.broadcasted_iota(jnp.int32, (1, 1, N_REL), 2)
    oh = (rid_ref[...][:, :, None] == iota_rel).astype(f32)
    oh = oh.reshape(BB * K_NBR, N_REL)
    re = jnp.dot(oh, reln, preferred_element_type=f32)

    g = jnp.tanh(hrb + re)
    prod = (g * tr).reshape(BB, K_NBR, E_DIM)
    logits = jnp.sum(prod, axis=2)

    lt = logits.T
    m = jnp.max(lt, axis=0, keepdims=True)
    e = jnp.exp(lt - m)
    attt = e / jnp.sum(e, axis=0, keepdims=True)
    att = attt.T
    nh = jnp.sum(t3n * att[:, :, None], axis=1)

    leaky = lambda x: jnp.where(x > 0, x, 0.2 * x)
    agg1 = leaky(dot_t(hn + nh, w1_ref[...]) + w1b_ref[...])
    agg2 = leaky(dot_t(hn * nh, w2_ref[...]) + w2b_ref[...])
    out_ref[...] = agg1 + agg2


def _tc_call(t3, hraw, rids, rel_embs, wr, wrb, w1, w1b, w2, w2b, seg):
    grid = t3.shape[0] // BB
    soff = seg * grid
    full = lambda i: (0, 0)
    return pl.pallas_call(
        _tc_body,
        grid=(grid,),
        in_specs=[
            pl.BlockSpec((BB, K_NBR, E_DIM), lambda i: (i, 0, 0)),
            pl.BlockSpec((BB, E_DIM), lambda i: (i, 0)),
            pl.BlockSpec((BB, K_NBR), lambda i: (soff + i, 0)),
            pl.BlockSpec((N_REL, E_DIM), full),
            pl.BlockSpec((E_DIM, E_DIM), full),
            pl.BlockSpec((1, E_DIM), full),
            pl.BlockSpec((E_DIM, E_DIM), full),
            pl.BlockSpec((1, E_DIM), full),
            pl.BlockSpec((E_DIM, E_DIM), full),
            pl.BlockSpec((1, E_DIM), full),
        ],
        out_specs=pl.BlockSpec((BB, E_DIM), lambda i: (i, 0)),
        out_shape=jax.ShapeDtypeStruct((t3.shape[0], E_DIM), jnp.float32),
        compiler_params=pltpu.CompilerParams(
            dimension_semantics=("arbitrary",),
        ),
    )(t3, hraw, rids, rel_embs, wr, wrb, w1, w1b, w2, w2b)


SEG = 4


@jax.jit
def kernel(idx, adj_ent, adj_rel, ent_embs, rel_embs,
           Wr_w, Wr_b, W1_w, W1_b, W2_w, W2_b):
    idx = jnp.clip(idx.astype(jnp.int32), 0, N_ENT - 1)
    idx2 = idx.reshape(BATCH // 128, 128)
    eids = _gather_adj(idx2, adj_ent)
    rids = _gather_adj(idx2, adj_rel)
    e2 = eids.reshape(BATCH * K_NBR // 128, 128)
    sb = BATCH // SEG
    outs = []
    for s in range(SEG):
        traw, hraw = _gather_t(e2, idx2, ent_embs, sb, s)
        t3 = traw.reshape(sb, K_NBR, E_DIM)
        outs.append(_tc_call(t3, hraw, rids, rel_embs,
                             Wr_w, Wr_b.reshape(1, E_DIM),
                             W1_w, W1_b.reshape(1, E_DIM),
                             W2_w, W2_b.reshape(1, E_DIM), s))
    return jnp.concatenate(outs, axis=0) if SEG > 1 else outs[0]
